# baseline (speedup 1.0000x reference)
import functools

import jax
import jax.numpy as jnp
from jax.experimental import pallas as pl
from jax.experimental.pallas import tpu as pltpu


def _round_up(x, m):
    return ((x + m - 1) // m) * m


def _core_kernel(emb_hbm, out_hbm, buf, in_sems, out_sems, *,
                 scale, core_chunks):
    pid = pl.program_id(0)

    def emit(chunks):
        def in_copy(i):
            base, rows = chunks[i]
            return pltpu.make_async_copy(
                emb_hbm.at[pl.ds(base, rows)],
                buf.at[i, pl.ds(0, rows)],
                in_sems.at[i],
            )

        def out_copy(i):
            base, rows = chunks[i]
            return pltpu.make_async_copy(
                buf.at[i, pl.ds(0, rows)],
                out_hbm.at[pl.ds(base, rows)],
                out_sems.at[i],
            )

        def body():
            n = len(chunks)
            for i in range(n):
                in_copy(i).start()
            for i in range(n):
                in_copy(i).wait()
                rows = chunks[i][1]
                buf[i, pl.ds(0, rows)] = (buf[i, pl.ds(0, rows)] * scale
                                          ).astype(buf.dtype)
                out_copy(i).start()
            for i in range(n):
                out_copy(i).wait()

        return body

    for c, chunks in enumerate(core_chunks):
        if chunks:
            pl.when(pid == c)(emit(chunks))


def kernel(x, embedding):
    max_seq_len, dim = embedding.shape
    seq_len = x.shape[1]
    if seq_len > max_seq_len:
        raise ValueError(f"seq_len={seq_len} exceeds max_seq_len={max_seq_len}")
    dtype = embedding.dtype
    itemsize = jnp.dtype(dtype).itemsize
    sub = max(8, 32 // itemsize)
    row_bytes = dim * itemsize

    n_cores = 2
    chunks_per_core = 2
    half_rows = max(sub, _round_up(-(-seq_len // n_cores), sub))
    chunk_rows = max(sub, _round_up(-(-half_rows // chunks_per_core), sub))

    core_chunks = []
    for c in range(n_cores):
        chunks = []
        base = c * half_rows
        end = min(seq_len, (c + 1) * half_rows)
        while base < end:
            rows = min(chunk_rows, end - base)
            chunks.append((base, rows))
            base += rows
        core_chunks.append(chunks)
    max_chunks = max(len(ch) for ch in core_chunks)

    return pl.pallas_call(
        functools.partial(_core_kernel, scale=float(dim) ** -0.5,
                          core_chunks=core_chunks),
        out_shape=jax.ShapeDtypeStruct((seq_len, dim), dtype),
        grid=(n_cores,),
        in_specs=[pl.BlockSpec(memory_space=pl.ANY)],
        out_specs=pl.BlockSpec(memory_space=pl.ANY),
        scratch_shapes=[
            pltpu.VMEM((max_chunks, chunk_rows, dim), dtype),
            pltpu.SemaphoreType.DMA((max_chunks,)),
            pltpu.SemaphoreType.DMA((max_chunks,)),
        ],
        compiler_params=pltpu.CompilerParams(
            dimension_semantics=("parallel",),
            vmem_limit_bytes=60000 * 1024,
        ),
    )(embedding)


# speedup vs baseline: 1.2943x; 1.2943x over previous
import functools

import jax
import jax.numpy as jnp
from jax.experimental import pallas as pl
from jax.experimental.pallas import tpu as pltpu


def _round_up(x, m):
    return ((x + m - 1) // m) * m


def _scale_kernel(emb_ref, out_ref, *, scale):
    out_ref[...] = (emb_ref[...] * scale).astype(out_ref.dtype)


def kernel(x, embedding):
    max_seq_len, dim = embedding.shape
    seq_len = x.shape[1]
    if seq_len > max_seq_len:
        raise ValueError(f"seq_len={seq_len} exceeds max_seq_len={max_seq_len}")
    dtype = embedding.dtype
    itemsize = jnp.dtype(dtype).itemsize
    sub = max(8, 32 // itemsize)
    row_bytes = dim * itemsize

    block_rows = max(sub, _round_up(-(-seq_len // 2), sub))
    num_blocks = pl.cdiv(seq_len, block_rows)

    block_bytes = block_rows * row_bytes
    vmem_limit = 96 * 1024 * 1024

    return pl.pallas_call(
        functools.partial(_scale_kernel, scale=float(dim) ** -0.5),
        out_shape=jax.ShapeDtypeStruct((seq_len, dim), dtype),
        grid=(num_blocks,),
        in_specs=[pl.BlockSpec((block_rows, dim), lambda i: (i, 0))],
        out_specs=pl.BlockSpec((block_rows, dim), lambda i: (i, 0)),
        compiler_params=pltpu.CompilerParams(
            dimension_semantics=("parallel",),
            vmem_limit_bytes=vmem_limit,
        ),
    )(embedding)
